# baseline (speedup 1.0000x reference)
import functools

import jax
import jax.numpy as jnp
from jax import lax
from jax.experimental import pallas as pl
from jax.experimental.pallas import tpu as pltpu
from jax.experimental.pallas import tpu_sc as plsc

ROWS = 128
COLS = 100000
NCORES = 2
NSUB = 16
NWORKERS = NCORES * NSUB
ROWS_PER_W = ROWS // NWORKERS
CHUNK = 10000
NCHUNKS = COLS // CHUNK
ITERS = CHUNK // 16
BIG = 1 << 24


def _argmax_body(x_hbm, out_hbm, buf, res, sem):
    wid = lax.axis_index("s") * NCORES + lax.axis_index("c")
    lanes = lax.iota(jnp.int32, 16)

    resv = jnp.zeros((16,), jnp.int32)

    for r in range(ROWS_PER_W):
        row_base = (wid * ROWS_PER_W + r) * COLS

        m = jnp.full((16,), -jnp.inf, jnp.float32)
        bi = jnp.zeros((16,), jnp.int32)
        for c in range(NCHUNKS):
            pltpu.sync_copy(
                x_hbm.at[pl.ds(row_base + c * CHUNK, CHUNK)], buf)

            def body(i, carry):
                m, bi = carry
                v = buf[pl.ds(i * 16, 16)]
                gt = v > m
                m = jnp.where(gt, v, m)
                bi = jnp.where(gt, c * ITERS + i, bi)
                return m, bi

            m, bi = lax.fori_loop(0, ITERS, body, (m, bi))

        row_max = jnp.max(m)
        glob = bi * 16 + lanes
        cand = jnp.where(m == row_max, glob, BIG)
        resv = jnp.where(lanes == r, jnp.min(cand), resv)

    res[...] = resv
    pltpu.sync_copy(res, out_hbm.at[pl.ds(wid * 16, 16)])


@functools.partial(jax.jit, static_argnames=())
def kernel(inputs):
    x = inputs.reshape(-1)
    mesh = plsc.VectorSubcoreMesh(
        core_axis_name="c", subcore_axis_name="s",
        num_cores=NCORES, num_subcores=NSUB)
    out = pl.kernel(
        _argmax_body,
        out_type=jax.ShapeDtypeStruct((NWORKERS * 16,), jnp.int32),
        mesh=mesh,
        compiler_params=pltpu.CompilerParams(needs_layout_passes=False),
        scratch_types=[
            pltpu.VMEM((CHUNK,), jnp.float32),
            pltpu.VMEM((16,), jnp.int32),
            pltpu.SemaphoreType.DMA,
        ],
    )(x)
    return out.reshape(NWORKERS, 16)[:, :ROWS_PER_W].reshape(-1)


# speedup vs baseline: 1.7393x; 1.7393x over previous
import functools

import jax
import jax.numpy as jnp
from jax import lax
from jax.experimental import pallas as pl
from jax.experimental.pallas import tpu as pltpu
from jax.experimental.pallas import tpu_sc as plsc

ROWS = 128
COLS = 100000
NCORES = 2
NSUB = 16
NWORKERS = NCORES * NSUB
ROWS_PER_W = ROWS // NWORKERS
CHUNK = 10000
NCHUNKS = COLS // CHUNK
VREGS = CHUNK // 16
ACCS = 5
UNROLL = 25
N_IT1 = VREGS // UNROLL
UNROLL2 = 5
N_IT2 = VREGS // UNROLL2
BIG = 1 << 24


def _argmax_body(x_hbm, out_hbm, bufs, res, sems):
    wid = lax.axis_index("s") * NCORES + lax.axis_index("c")
    lanes = lax.iota(jnp.int32, 16)
    resv = jnp.zeros((16,), jnp.int32)
    neg_inf = jnp.full((16,), -jnp.inf, jnp.float32)

    for r in range(ROWS_PER_W):
        row_base = (wid * ROWS_PER_W + r) * COLS

        copies = [
            pltpu.async_copy(
                x_hbm.at[pl.ds(row_base + c * CHUNK, CHUNK)],
                bufs.at[pl.ds(c * CHUNK, CHUNK)], sems.at[c])
            for c in range(NCHUNKS)
        ]

        row_max = -jnp.inf
        best_chunk = jnp.int32(0)
        for c in range(NCHUNKS):
            copies[c].wait()

            def body1(i, ms):
                ms = list(ms)
                for u in range(UNROLL):
                    v = bufs[pl.ds(c * CHUNK + (i * UNROLL + u) * 16, 16)]
                    ms[u % ACCS] = jnp.maximum(ms[u % ACCS], v)
                return tuple(ms)

            ms = lax.fori_loop(0, N_IT1, body1, (neg_inf,) * ACCS)
            mm = jnp.maximum(jnp.maximum(ms[0], ms[1]),
                             jnp.maximum(ms[2], jnp.maximum(ms[3], ms[4])))
            cm = jnp.max(mm)
            best_chunk = jnp.where(cm > row_max, c, best_chunk)
            row_max = jnp.maximum(row_max, cm)

        def body2(i, bis):
            bis = list(bis)
            for u in range(UNROLL2):
                it = i * UNROLL2 + u
                v = bufs[pl.ds(best_chunk * CHUNK + it * 16, 16)]
                cand = jnp.where(v == row_max, it, BIG)
                bis[u] = jnp.minimum(bis[u], cand)
            return tuple(bis)

        big_v = jnp.full((16,), BIG, jnp.int32)
        bis = lax.fori_loop(0, N_IT2, body2, (big_v,) * UNROLL2)
        bi = jnp.minimum(jnp.minimum(bis[0], bis[1]),
                         jnp.minimum(bis[2], jnp.minimum(bis[3], bis[4])))
        glob = best_chunk * CHUNK + bi * 16 + lanes
        resv = jnp.where(lanes == r, jnp.min(glob), resv)

    res[...] = resv
    pltpu.sync_copy(res, out_hbm.at[pl.ds(wid * 16, 16)])


@functools.partial(jax.jit, static_argnames=())
def kernel(inputs):
    x = inputs.reshape(-1)
    mesh = plsc.VectorSubcoreMesh(
        core_axis_name="c", subcore_axis_name="s",
        num_cores=NCORES, num_subcores=NSUB)
    out = pl.kernel(
        _argmax_body,
        out_type=jax.ShapeDtypeStruct((NWORKERS * 16,), jnp.int32),
        mesh=mesh,
        compiler_params=pltpu.CompilerParams(needs_layout_passes=False),
        scratch_types=[
            pltpu.VMEM((COLS,), jnp.float32),
            pltpu.VMEM((16,), jnp.int32),
            pltpu.SemaphoreType.DMA((NCHUNKS,)),
        ],
    )(x)
    return out.reshape(NWORKERS, 16)[:, :ROWS_PER_W].reshape(-1)


# speedup vs baseline: 2.3796x; 1.3682x over previous
import functools

import jax
import jax.numpy as jnp
from jax import lax
from jax.experimental import pallas as pl
from jax.experimental.pallas import tpu as pltpu
from jax.experimental.pallas import tpu_sc as plsc

ROWS = 128
COLS = 100000
NCORES = 2
NSUB = 16
CW = 1408
NFULL = 71
TAIL_OFF = NFULL * CW
TAIL_W = COLS - TAIL_OFF
NCH = 36
NB = 4
VR = CW // 16
ACCS = 4
N_IT1 = VR // ACCS
BIG = 1 << 24


def _argmax_body(x_hbm, out_hbm, b0, b1, b2, b3, tailb, mbuf, gbuf, res,
                 shared_m, shared_g, sems, tail_sem):
    bufs = (b0, b1, b2, b3)
    cidx = lax.axis_index("c")
    s = lax.axis_index("s")
    g = cidx * 8 + lax.rem(s, 8)
    h = s // 8
    base_k = h * 35
    lanes = lax.iota(jnp.int32, 16)

    def chunk_src(kk):
        return x_hbm.at[pl.ds(g * 8, 8), pl.ds(kk * CW, CW)]

    for b in range(NB):
        pltpu.async_copy(chunk_src(base_k + b), bufs[b], sems.at[b])
    tail_copy = pltpu.async_copy(
        x_hbm.at[pl.ds(g * 8, 8), pl.ds(TAIL_OFF, TAIL_W)], tailb, tail_sem)

    def process_rows(buf, colbase, nvr, carry):
        carry = list(carry)
        for rr in range(8):
            neg = jnp.full((16,), -jnp.inf, jnp.float32)
            if nvr >= ACCS:
                def body1(i, ms):
                    ms = list(ms)
                    for u in range(ACCS):
                        v = buf[rr, pl.ds((i * ACCS + u) * 16, 16)]
                        ms[u] = jnp.maximum(ms[u], v)
                    return tuple(ms)

                ms = lax.fori_loop(0, nvr // ACCS, body1, (neg,) * ACCS)
                mm = jnp.maximum(jnp.maximum(ms[0], ms[1]),
                                 jnp.maximum(ms[2], ms[3]))
            else:
                mm = neg
                for j in range(nvr):
                    mm = jnp.maximum(mm, buf[rr, pl.ds(j * 16, 16)])
            cm = jnp.max(mm)

            def rescan():
                if nvr >= ACCS:
                    def body2(i, bi):
                        v = buf[rr, pl.ds(i * 16, 16)]
                        return jnp.minimum(bi, jnp.where(v == cm, i, BIG))

                    bi = lax.fori_loop(
                        0, nvr, body2, jnp.full((16,), BIG, jnp.int32))
                else:
                    bi = jnp.full((16,), BIG, jnp.int32)
                    for j in range(nvr):
                        v = buf[rr, pl.ds(j * 16, 16)]
                        bi = jnp.minimum(bi, jnp.where(v == cm, j, BIG))
                return jnp.min(colbase + bi * 16 + lanes)

            pred = cm > carry[rr]
            carry[8 + rr] = lax.cond(pred, rescan, lambda: carry[8 + rr])
            carry[rr] = jnp.maximum(carry[rr], cm)
        return tuple(carry)

    def outer(k4, carry):
        for b in range(NB):
            kk = base_k + k4 * NB + b
            pltpu.make_async_copy(chunk_src(kk), bufs[b], sems.at[b]).wait()
            carry = process_rows(bufs[b], kk * CW, VR, carry)

            @pl.when(k4 < (NCH // NB) - 1)
            def _():
                pltpu.async_copy(
                    chunk_src(kk + NB), bufs[b], sems.at[b])
        return carry

    init = (-jnp.inf,) * 8 + (jnp.int32(0),) * 8
    carry = lax.fori_loop(0, NCH // NB, outer, init)

    tail_copy.wait()
    carry = process_rows(tailb, TAIL_OFF, TAIL_W // 16, carry)

    mvec = jnp.full((16,), -jnp.inf, jnp.float32)
    gvec = jnp.zeros((16,), jnp.int32)
    for rr in range(8):
        mvec = jnp.where(lanes == rr, carry[rr], mvec)
        gvec = jnp.where(lanes == rr, carry[8 + rr], gvec)
    mbuf[...] = mvec
    gbuf[...] = gvec
    pltpu.sync_copy(mbuf, shared_m.at[pl.ds(s * 16, 16)])
    pltpu.sync_copy(gbuf, shared_g.at[pl.ds(s * 16, 16)])
    plsc.subcore_barrier()

    @pl.when(h == 0)
    def _():
        pltpu.sync_copy(shared_m.at[pl.ds((s + 8) * 16, 16)], mbuf)
        pltpu.sync_copy(shared_g.at[pl.ds((s + 8) * 16, 16)], gbuf)
        m1 = mbuf[...]
        i1 = gbuf[...]
        better = (m1 > mvec) | ((m1 == mvec) & (i1 < gvec))
        res[...] = jnp.where(better, i1, gvec)
        pltpu.sync_copy(res.at[pl.ds(0, 8)], out_hbm.at[pl.ds(g * 8, 8)])


@functools.partial(jax.jit, static_argnames=())
def kernel(inputs):
    mesh = plsc.VectorSubcoreMesh(
        core_axis_name="c", subcore_axis_name="s",
        num_cores=NCORES, num_subcores=NSUB)
    return pl.kernel(
        _argmax_body,
        out_type=jax.ShapeDtypeStruct((ROWS,), jnp.int32),
        mesh=mesh,
        compiler_params=pltpu.CompilerParams(needs_layout_passes=False),
        scratch_types=[
            pltpu.VMEM((8, CW), jnp.float32),
            pltpu.VMEM((8, CW), jnp.float32),
            pltpu.VMEM((8, CW), jnp.float32),
            pltpu.VMEM((8, CW), jnp.float32),
            pltpu.VMEM((8, TAIL_W), jnp.float32),
            pltpu.VMEM((16,), jnp.float32),
            pltpu.VMEM((16,), jnp.int32),
            pltpu.VMEM((16,), jnp.int32),
            pltpu.VMEM_SHARED((256,), jnp.float32),
            pltpu.VMEM_SHARED((256,), jnp.int32),
            pltpu.SemaphoreType.DMA((NB,)),
            pltpu.SemaphoreType.DMA,
        ],
    )(inputs)


# speedup vs baseline: 3.4580x; 1.4532x over previous
import functools

import jax
import jax.numpy as jnp
from jax import lax
from jax.experimental import pallas as pl
from jax.experimental.pallas import tpu as pltpu
from jax.experimental.pallas import tpu_sc as plsc

ROWS = 128
COLS = 100000
NCORES = 2
NSUB = 16
CW = 1408
NFULL = 71
TAIL_OFF = NFULL * CW
TAIL_W = COLS - TAIL_OFF
NCH = 36
NB = 4
VR = CW // 16
UNR = 4
N_IT = VR // UNR
BIG = 1 << 24


def _argmax_body(x_hbm, out_hbm, b0, b1, b2, b3, tailb, mbuf, gbuf, res,
                 shared_m, shared_g, sems, tail_sem):
    bufs = (b0, b1, b2, b3)
    cidx = lax.axis_index("c")
    s = lax.axis_index("s")
    g = cidx * 8 + lax.rem(s, 8)
    h = s // 8
    base_k = h * 35
    lanes = lax.iota(jnp.int32, 16)

    def chunk_src(kk):
        return x_hbm.at[pl.ds(g * 8, 8), pl.ds(kk * CW, CW)]

    for b in range(NB):
        pltpu.async_copy(chunk_src(base_k + b), bufs[b], sems.at[b])
    tail_copy = pltpu.async_copy(
        x_hbm.at[pl.ds(g * 8, 8), pl.ds(TAIL_OFF, TAIL_W)], tailb, tail_sem)

    def process(buf, colbase, nvr, unr, carry):
        def step(j, cv, carry):
            ms, gs = list(carry[:8]), list(carry[8:])
            for rr in range(8):
                v = buf[rr, pl.ds(j * 16, 16)]
                gt = v > ms[rr]
                ms[rr] = jnp.maximum(ms[rr], v)
                gs[rr] = jnp.where(gt, cv, gs[rr])
            return tuple(ms) + tuple(gs)

        if nvr <= unr:
            for j in range(nvr):
                carry = step(j, colbase + j * 16 + lanes, carry)
            return carry

        def body(i, carry):
            for u in range(unr):
                j = i * unr + u
                carry = step(j, colbase + j * 16 + lanes, carry)
            return carry

        return lax.fori_loop(0, nvr // unr, body, carry)

    def outer(k4, carry):
        for b in range(NB):
            kk = base_k + k4 * NB + b
            pltpu.make_async_copy(chunk_src(kk), bufs[b], sems.at[b]).wait()
            carry = process(bufs[b], kk * CW, VR, UNR, carry)

            @pl.when(k4 < (NCH // NB) - 1)
            def _():
                pltpu.async_copy(chunk_src(kk + NB), bufs[b], sems.at[b])
        return carry

    neg = jnp.full((16,), -jnp.inf, jnp.float32)
    zero = jnp.zeros((16,), jnp.int32)
    init = (neg,) * 8 + (zero,) * 8
    carry = lax.fori_loop(0, NCH // NB, outer, init)

    tail_copy.wait()
    carry = process(tailb, TAIL_OFF, TAIL_W // 16, UNR, carry)

    mvec = neg
    gvec = zero
    for rr in range(8):
        m = carry[rr]
        rm = jnp.max(m)
        cand = jnp.where(m == rm, carry[8 + rr], BIG)
        mvec = jnp.where(lanes == rr, rm, mvec)
        gvec = jnp.where(lanes == rr, jnp.min(cand), gvec)

    mbuf[...] = mvec
    gbuf[...] = gvec
    pltpu.sync_copy(mbuf, shared_m.at[pl.ds(s * 16, 16)])
    pltpu.sync_copy(gbuf, shared_g.at[pl.ds(s * 16, 16)])
    plsc.subcore_barrier()

    @pl.when(h == 0)
    def _():
        pltpu.sync_copy(shared_m.at[pl.ds((s + 8) * 16, 16)], mbuf)
        pltpu.sync_copy(shared_g.at[pl.ds((s + 8) * 16, 16)], gbuf)
        m1 = mbuf[...]
        i1 = gbuf[...]
        better = (m1 > mvec) | ((m1 == mvec) & (i1 < gvec))
        res[...] = jnp.where(better, i1, gvec)
        pltpu.sync_copy(res.at[pl.ds(0, 8)], out_hbm.at[pl.ds(g * 8, 8)])


@functools.partial(jax.jit, static_argnames=())
def kernel(inputs):
    mesh = plsc.VectorSubcoreMesh(
        core_axis_name="c", subcore_axis_name="s",
        num_cores=NCORES, num_subcores=NSUB)
    return pl.kernel(
        _argmax_body,
        out_type=jax.ShapeDtypeStruct((ROWS,), jnp.int32),
        mesh=mesh,
        compiler_params=pltpu.CompilerParams(needs_layout_passes=False),
        scratch_types=[
            pltpu.VMEM((8, CW), jnp.float32),
            pltpu.VMEM((8, CW), jnp.float32),
            pltpu.VMEM((8, CW), jnp.float32),
            pltpu.VMEM((8, CW), jnp.float32),
            pltpu.VMEM((8, TAIL_W), jnp.float32),
            pltpu.VMEM((16,), jnp.float32),
            pltpu.VMEM((16,), jnp.int32),
            pltpu.VMEM((16,), jnp.int32),
            pltpu.VMEM_SHARED((256,), jnp.float32),
            pltpu.VMEM_SHARED((256,), jnp.int32),
            pltpu.SemaphoreType.DMA((NB,)),
            pltpu.SemaphoreType.DMA,
        ],
    )(inputs)


# speedup vs baseline: 5.9413x; 1.7181x over previous
import functools

import jax
import jax.numpy as jnp
from jax import lax
from jax.experimental import pallas as pl
from jax.experimental.pallas import tpu as pltpu
from jax.experimental.pallas import tpu_sc as plsc

ROWS = 128
COLS = 100000
NCORES = 2
NSUB = 16
NW = NCORES * NSUB
SPAN = 3128
LAST_OFF = COLS - SPAN
CR = 64
NC = -(-SPAN // CR)
LAST_COFF = SPAN - CR
NB = 4
UNR = 2
BIG = 1 << 24


def _pass1_body(xt_hbm, outm_hbm, outg_hbm, b0, b1, b2, b3, mres, gres, sems):
    bufs = (b0, b1, b2, b3)
    cidx = lax.axis_index("c")
    s = lax.axis_index("s")
    w = s * NCORES + cidx
    voff = jnp.minimum(w * SPAN, LAST_OFF)

    def chunk_src(coff):
        return xt_hbm.at[pl.ds(voff + coff, CR), :]

    def coff_of(k):
        return jnp.minimum(k * CR, LAST_COFF)

    for b in range(NB):
        pltpu.async_copy(chunk_src(coff_of(b)), bufs[b], sems.at[b])

    def process(buf, coff, carry):
        def body(i, carry):
            ms, gs = list(carry[:8]), list(carry[8:])
            for u in range(UNR):
                v_idx = i * UNR + u
                vid = voff + coff + v_idx
                for q in range(8):
                    v = buf[v_idx, pl.ds(q * 16, 16)]
                    gt = v > ms[q]
                    ms[q] = jnp.maximum(ms[q], v)
                    gs[q] = jnp.where(gt, vid, gs[q])
            return tuple(ms) + tuple(gs)

        return lax.fori_loop(0, CR // UNR, body, carry)

    def outer(k4, carry):
        for b in range(NB):
            k = k4 * NB + b
            coff = coff_of(k)
            pltpu.make_async_copy(chunk_src(coff), bufs[b], sems.at[b]).wait()
            carry = process(bufs[b], coff, carry)

            @pl.when(k + NB < NC)
            def _():
                pltpu.async_copy(chunk_src(coff_of(k + NB)), bufs[b],
                                 sems.at[b])
        return carry

    neg = jnp.full((16,), -jnp.inf, jnp.float32)
    zero = jnp.zeros((16,), jnp.int32)
    carry = lax.fori_loop(0, NC // NB, outer, (neg,) * 8 + (zero,) * 8)
    for k in range((NC // NB) * NB, NC):
        b = k % NB
        coff = coff_of(k)
        pltpu.make_async_copy(chunk_src(coff), bufs[b], sems.at[b]).wait()
        carry = process(bufs[b], coff, carry)

    for q in range(8):
        mres[pl.ds(q * 16, 16)] = carry[q]
        gres[pl.ds(q * 16, 16)] = carry[8 + q]
    pltpu.sync_copy(mres, outm_hbm.at[pl.ds(w * ROWS, ROWS)])
    pltpu.sync_copy(gres, outg_hbm.at[pl.ds(w * ROWS, ROWS)])


def _merge_body(pm_hbm, pg_hbm, out_hbm, mv, gv, res, sem):
    cidx = lax.axis_index("c")
    s = lax.axis_index("s")
    w = s * NCORES + cidx
    lanes = lax.iota(jnp.int32, 16)

    pltpu.async_copy(pm_hbm, mv, sem).wait()
    pltpu.async_copy(pg_hbm, gv, sem).wait()

    resv = jnp.zeros((16,), jnp.int32)
    for q in range(4):
        r = w * 4 + q
        idx0 = lanes * ROWS + r
        idx1 = idx0 + 16 * ROWS
        m0 = plsc.load_gather(mv, [idx0])
        m1 = plsc.load_gather(mv, [idx1])
        g0 = plsc.load_gather(gv, [idx0])
        g1 = plsc.load_gather(gv, [idx1])
        mm = jnp.max(jnp.maximum(m0, m1))
        c0 = jnp.where(m0 == mm, g0, BIG)
        c1 = jnp.where(m1 == mm, g1, BIG)
        best = jnp.min(jnp.minimum(c0, c1))
        resv = jnp.where(lanes == q, best, resv)
    res[...] = resv
    pltpu.sync_copy(res.at[pl.ds(0, 8)], out_hbm.at[pl.ds(w * 8, 8)])


@functools.partial(jax.jit, static_argnames=())
def kernel(inputs):
    xt = inputs.T
    mesh = plsc.VectorSubcoreMesh(
        core_axis_name="c", subcore_axis_name="s",
        num_cores=NCORES, num_subcores=NSUB)
    cp = pltpu.CompilerParams(needs_layout_passes=False)
    pm, pg = pl.kernel(
        _pass1_body,
        out_type=(jax.ShapeDtypeStruct((NW * ROWS,), jnp.float32),
                  jax.ShapeDtypeStruct((NW * ROWS,), jnp.int32)),
        mesh=mesh,
        compiler_params=cp,
        scratch_types=[
            pltpu.VMEM((CR, ROWS), jnp.float32),
            pltpu.VMEM((CR, ROWS), jnp.float32),
            pltpu.VMEM((CR, ROWS), jnp.float32),
            pltpu.VMEM((CR, ROWS), jnp.float32),
            pltpu.VMEM((ROWS,), jnp.float32),
            pltpu.VMEM((ROWS,), jnp.int32),
            pltpu.SemaphoreType.DMA((NB,)),
        ],
    )(xt)
    out = pl.kernel(
        _merge_body,
        out_type=jax.ShapeDtypeStruct((NW * 8,), jnp.int32),
        mesh=mesh,
        compiler_params=cp,
        scratch_types=[
            pltpu.VMEM((NW * ROWS,), jnp.float32),
            pltpu.VMEM((NW * ROWS,), jnp.int32),
            pltpu.VMEM((16,), jnp.int32),
            pltpu.SemaphoreType.DMA,
        ],
    )(pm, pg)
    return out.reshape(NW, 8)[:, :4].reshape(-1)


# speedup vs baseline: 6.0991x; 1.0266x over previous
import functools

import jax
import jax.numpy as jnp
from jax import lax
from jax.experimental import pallas as pl
from jax.experimental.pallas import tpu as pltpu
from jax.experimental.pallas import tpu_sc as plsc

ROWS = 128
COLS = 100000
NCORES = 2
NSUB = 16
NW = NCORES * NSUB
SPAN = 3128
LAST_OFF = COLS - SPAN
CR = 64
NC = -(-SPAN // CR)
LAST_COFF = SPAN - CR
NB = 4
UNR = 4
BIG = 1 << 24


def _pass1_body(xt_hbm, outm_hbm, outg_hbm, b0, b1, b2, b3, mres, gres, sems):
    bufs = (b0, b1, b2, b3)
    cidx = lax.axis_index("c")
    s = lax.axis_index("s")
    w = s * NCORES + cidx
    voff = jnp.minimum(w * SPAN, LAST_OFF)

    def chunk_src(coff):
        return xt_hbm.at[pl.ds(voff + coff, CR), :]

    def coff_of(k):
        return jnp.minimum(k * CR, LAST_COFF)

    for b in range(NB):
        pltpu.async_copy(chunk_src(coff_of(b)), bufs[b], sems.at[b])

    def process(buf, coff, carry):
        def body(i, carry):
            ms, gs = list(carry[:8]), list(carry[8:])
            for u in range(UNR):
                v_idx = i * UNR + u
                vid = voff + coff + v_idx
                for q in range(8):
                    v = buf[v_idx, pl.ds(q * 16, 16)]
                    gt = v > ms[q]
                    ms[q] = jnp.maximum(ms[q], v)
                    gs[q] = jnp.where(gt, vid, gs[q])
            return tuple(ms) + tuple(gs)

        return lax.fori_loop(0, CR // UNR, body, carry)

    def outer(k4, carry):
        for b in range(NB):
            k = k4 * NB + b
            coff = coff_of(k)
            pltpu.make_async_copy(chunk_src(coff), bufs[b], sems.at[b]).wait()
            carry = process(bufs[b], coff, carry)

            @pl.when(k + NB < NC)
            def _():
                pltpu.async_copy(chunk_src(coff_of(k + NB)), bufs[b],
                                 sems.at[b])
        return carry

    neg = jnp.full((16,), -jnp.inf, jnp.float32)
    zero = jnp.zeros((16,), jnp.int32)
    carry = lax.fori_loop(0, NC // NB, outer, (neg,) * 8 + (zero,) * 8)
    for k in range((NC // NB) * NB, NC):
        b = k % NB
        coff = coff_of(k)
        pltpu.make_async_copy(chunk_src(coff), bufs[b], sems.at[b]).wait()
        carry = process(bufs[b], coff, carry)

    for q in range(8):
        mres[pl.ds(q * 16, 16)] = carry[q]
        gres[pl.ds(q * 16, 16)] = carry[8 + q]
    pltpu.sync_copy(mres, outm_hbm.at[pl.ds(w * ROWS, ROWS)])
    pltpu.sync_copy(gres, outg_hbm.at[pl.ds(w * ROWS, ROWS)])


def _merge_body(pm_hbm, pg_hbm, out_hbm, mv, gv, res, sem):
    cidx = lax.axis_index("c")
    s = lax.axis_index("s")
    lanes = lax.iota(jnp.int32, 16)

    pltpu.async_copy(pm_hbm, mv, sem).wait()
    pltpu.async_copy(pg_hbm, gv, sem).wait()

    resv = jnp.zeros((16,), jnp.int32)
    for q in range(8):
        r = s * 8 + q
        idx0 = lanes * ROWS + r
        idx1 = idx0 + 16 * ROWS
        m0 = plsc.load_gather(mv, [idx0])
        m1 = plsc.load_gather(mv, [idx1])
        g0 = plsc.load_gather(gv, [idx0])
        g1 = plsc.load_gather(gv, [idx1])
        mm = jnp.max(jnp.maximum(m0, m1))
        c0 = jnp.where(m0 == mm, g0, BIG)
        c1 = jnp.where(m1 == mm, g1, BIG)
        best = jnp.min(jnp.minimum(c0, c1))
        resv = jnp.where(lanes == q, best, resv)
    res[...] = resv

    @pl.when(cidx == 0)
    def _():
        pltpu.sync_copy(res.at[pl.ds(0, 8)], out_hbm.at[pl.ds(s * 8, 8)])


@functools.partial(jax.jit, static_argnames=())
def kernel(inputs):
    xt = inputs.T
    mesh = plsc.VectorSubcoreMesh(
        core_axis_name="c", subcore_axis_name="s",
        num_cores=NCORES, num_subcores=NSUB)
    cp = pltpu.CompilerParams(needs_layout_passes=False)
    pm, pg = pl.kernel(
        _pass1_body,
        out_type=(jax.ShapeDtypeStruct((NW * ROWS,), jnp.float32),
                  jax.ShapeDtypeStruct((NW * ROWS,), jnp.int32)),
        mesh=mesh,
        compiler_params=cp,
        scratch_types=[
            pltpu.VMEM((CR, ROWS), jnp.float32),
            pltpu.VMEM((CR, ROWS), jnp.float32),
            pltpu.VMEM((CR, ROWS), jnp.float32),
            pltpu.VMEM((CR, ROWS), jnp.float32),
            pltpu.VMEM((ROWS,), jnp.float32),
            pltpu.VMEM((ROWS,), jnp.int32),
            pltpu.SemaphoreType.DMA((NB,)),
        ],
    )(xt)
    out = pl.kernel(
        _merge_body,
        out_type=jax.ShapeDtypeStruct((ROWS,), jnp.int32),
        mesh=mesh,
        compiler_params=cp,
        scratch_types=[
            pltpu.VMEM((NW * ROWS,), jnp.float32),
            pltpu.VMEM((NW * ROWS,), jnp.int32),
            pltpu.VMEM((16,), jnp.int32),
            pltpu.SemaphoreType.DMA,
        ],
    )(pm, pg)
    return out
